# Initial kernel scaffold; baseline (speedup 1.0000x reference)
import jax
import jax.numpy as jnp
from jax.experimental import pallas as pl


def kernel(x, edge_index, edge_attr, en1_W1, en1_b1, en1_W2, en1_b2, en2_W1, en2_b1, en2_W2, en2_b2, root1, bias1, root2, bias2, proj_W, proj_b):
    raise NotImplementedError("write your pallas kernel here")


# speedup vs baseline: 2.5524x; 2.5524x over previous
import functools

import jax
import jax.numpy as jnp
from jax import lax
from jax.experimental import pallas as pl
from jax.experimental.pallas import tpu as pltpu
from jax.experimental.pallas import tpu_sc as plsc

_NC = 2
_NS = 16
_NW = _NC * _NS
_CH = 128


def _sc_mesh():
    return plsc.VectorSubcoreMesh(core_axis_name="c", subcore_axis_name="s")


def _gather_rows(table, idx2d, e_pad, d):
    chunks_w = e_pad // _NW // _CH

    @functools.partial(
        pl.kernel,
        out_type=jax.ShapeDtypeStruct((e_pad, d), jnp.float32),
        mesh=_sc_mesh(),
        scratch_types=[
            pltpu.VMEM((chunks_w, _CH), jnp.int32),
            pltpu.VMEM((_CH, d), jnp.float32),
            pltpu.VMEM((_CH, d), jnp.float32),
            pltpu.SemaphoreType.DMA,
            pltpu.SemaphoreType.DMA,
        ],
        compiler_params=pltpu.CompilerParams(use_tc_tiling_on_sc=False),
    )
    def gk(table_h, idx_h, out_h, idx_v, buf0, buf1, sem0, sem1):
        wid = lax.axis_index("s") * _NC + lax.axis_index("c")
        cbase = wid * chunks_w
        pltpu.sync_copy(idx_h.at[pl.ds(cbase, chunks_w)], idx_v)

        def body(jj, carry):
            j0 = 2 * jj
            j1 = 2 * jj + 1
            d0 = pltpu.async_copy(table_h.at[idx_v.at[j0]], buf0, sem0)
            d1 = pltpu.async_copy(table_h.at[idx_v.at[j1]], buf1, sem1)
            d0.wait()
            pltpu.sync_copy(buf0, out_h.at[pl.ds((cbase + j0) * _CH, _CH)])
            d1.wait()
            pltpu.sync_copy(buf1, out_h.at[pl.ds((cbase + j1) * _CH, _CH)])
            return carry

        lax.fori_loop(0, chunks_w // 2, body, 0)

    return gk(table, idx2d)


def _scatter_rows(msg, idx2d, e_pad, d, n_acc, with_cnt):
    chunks_w = e_pad // _NW // _CH
    rows_t = n_acc // _NS

    z32 = jnp.zeros((n_acc, d), jnp.float32)
    out_types = [jax.ShapeDtypeStruct((_NC, n_acc, d), jnp.float32)]
    scratch = [
        pltpu.VMEM((chunks_w, _CH), jnp.int32),
        pltpu.VMEM((_CH, d), jnp.float32),
        pltpu.VMEM_SHARED((n_acc, d), jnp.float32),
        pltpu.SemaphoreType.DMA,
    ]
    extra_in = []
    if with_cnt:
        extra_in = [jnp.zeros((n_acc, 16), jnp.float32),
                    jnp.ones((_CH, 16), jnp.float32)]
        out_types.append(jax.ShapeDtypeStruct((_NC, n_acc, 16), jnp.float32))
        scratch += [
            pltpu.VMEM((_CH, 16), jnp.float32),
            pltpu.VMEM_SHARED((n_acc, 16), jnp.float32),
            pltpu.SemaphoreType.DMA,
        ]

    @functools.partial(
        pl.kernel,
        out_type=tuple(out_types) if with_cnt else out_types[0],
        mesh=_sc_mesh(),
        scratch_types=scratch,
        compiler_params=pltpu.CompilerParams(use_tc_tiling_on_sc=False),
    )
    def sk(*refs):
        if with_cnt:
            (msg_h, idx_h, z32_h, z16_h, ones_h, out_h, cnt_h,
             idx_v, msg_v, acc_s, sem_a, ones_v, cnt_s, sem_c) = refs
        else:
            (msg_h, idx_h, z32_h, out_h,
             idx_v, msg_v, acc_s, sem_a) = refs
        cid = lax.axis_index("c")
        sid = lax.axis_index("s")
        wid = sid * _NC + cid
        rs = sid * rows_t
        pltpu.sync_copy(z32_h.at[pl.ds(rs, rows_t)], acc_s.at[pl.ds(rs, rows_t)])
        if with_cnt:
            pltpu.sync_copy(z16_h.at[pl.ds(rs, rows_t)], cnt_s.at[pl.ds(rs, rows_t)])
            pltpu.sync_copy(ones_h, ones_v)
        plsc.subcore_barrier()
        cbase = wid * chunks_w
        pltpu.sync_copy(idx_h.at[pl.ds(cbase, chunks_w)], idx_v)

        def body(j, carry):
            pltpu.sync_copy(msg_h.at[pl.ds((cbase + j) * _CH, _CH)], msg_v)
            pltpu.async_copy(msg_v, acc_s.at[idx_v.at[j]], sem_a, add=True).wait()
            if with_cnt:
                pltpu.async_copy(ones_v, cnt_s.at[idx_v.at[j]], sem_c, add=True).wait()
            return carry

        lax.fori_loop(0, chunks_w, body, 0)
        plsc.subcore_barrier()
        pltpu.sync_copy(acc_s.at[pl.ds(rs, rows_t)],
                        out_h.at[cid].at[pl.ds(rs, rows_t)])
        if with_cnt:
            pltpu.sync_copy(cnt_s.at[pl.ds(rs, rows_t)],
                            cnt_h.at[cid].at[pl.ds(rs, rows_t)])

    if with_cnt:
        return sk(msg, idx2d, z32, *extra_in)
    return sk(msg, idx2d, z32)


def _edge_msgs(ea_pad, xj, w1, b1, p_mat, b_mat, e_pad):
    eb = 512
    d_e = ea_pad.shape[1]
    d_in = xj.shape[1]
    d_out = p_mat.shape[1]
    kd = p_mat.shape[0]
    h_dim = kd // d_in

    w1rep = jnp.repeat(w1, d_in, axis=1)
    b1rep = jnp.repeat(b1, d_in).reshape(1, kd)

    def ek(ea_ref, xj_ref, w1_ref, b1_ref, p_ref, b_ref, out_ref):
        th = jnp.maximum(
            jnp.dot(ea_ref[...], w1_ref[...], preferred_element_type=jnp.float32)
            + b1_ref[...], 0.0)
        xjv = xj_ref[...]
        xt = jnp.concatenate([xjv] * h_dim, axis=1)
        out_ref[...] = (
            jnp.dot(th * xt, p_ref[...], preferred_element_type=jnp.float32)
            + jnp.dot(xjv, b_ref[...], preferred_element_type=jnp.float32))

    return pl.pallas_call(
        ek,
        grid=(e_pad // eb,),
        in_specs=[
            pl.BlockSpec((eb, d_e), lambda i: (i, 0)),
            pl.BlockSpec((eb, d_in), lambda i: (i, 0)),
            pl.BlockSpec((d_e, kd), lambda i: (0, 0)),
            pl.BlockSpec((1, kd), lambda i: (0, 0)),
            pl.BlockSpec((kd, d_out), lambda i: (0, 0)),
            pl.BlockSpec((d_in, d_out), lambda i: (0, 0)),
        ],
        out_specs=pl.BlockSpec((eb, d_out), lambda i: (i, 0)),
        out_shape=jax.ShapeDtypeStruct((e_pad, d_out), jnp.float32),
    )(ea_pad, xj, w1rep, b1rep, p_mat, b_mat)


def _node_update(parts, cnts, x, root, bias, n):
    nb = 1000
    d_in = x.shape[1]
    d = root.shape[1]

    def nk(p_ref, c_ref, x_ref, r_ref, b_ref, out_ref):
        s = p_ref[0] + p_ref[1]
        cnt = jnp.maximum(c_ref[0, :, 0:1] + c_ref[1, :, 0:1], 1.0)
        out_ref[...] = jnp.maximum(
            s / cnt
            + jnp.dot(x_ref[...], r_ref[...], preferred_element_type=jnp.float32)
            + b_ref[...], 0.0)

    return pl.pallas_call(
        nk,
        grid=(n // nb,),
        in_specs=[
            pl.BlockSpec((2, nb, d), lambda i: (0, i, 0)),
            pl.BlockSpec((2, nb, 16), lambda i: (0, i, 0)),
            pl.BlockSpec((nb, d_in), lambda i: (i, 0)),
            pl.BlockSpec((d_in, d), lambda i: (0, 0)),
            pl.BlockSpec((1, d), lambda i: (0, 0)),
        ],
        out_specs=pl.BlockSpec((nb, d), lambda i: (i, 0)),
        out_shape=jax.ShapeDtypeStruct((n, d), jnp.float32),
    )(parts, cnts, x, root, bias.reshape(1, d))


def _node_pool_proj(parts, cnts, x, root, bias, proj_w, proj_b, n):
    nb = 1000
    nblk = n // nb
    d_in = x.shape[1]
    d = root.shape[1]
    d_out = proj_w.shape[1]

    def nk(p_ref, c_ref, x_ref, r_ref, b_ref, pw_ref, pb_ref, out_ref, acc_ref):
        i = pl.program_id(0)
        s = p_ref[0] + p_ref[1]
        cnt = jnp.maximum(c_ref[0, :, 0:1] + c_ref[1, :, 0:1], 1.0)
        h2 = jnp.maximum(
            s / cnt
            + jnp.dot(x_ref[...], r_ref[...], preferred_element_type=jnp.float32)
            + b_ref[...], 0.0)
        part = jnp.sum(h2, axis=0, keepdims=True)

        @pl.when(i == 0)
        def _():
            acc_ref[...] = part

        @pl.when(i > 0)
        def _():
            acc_ref[...] = acc_ref[...] + part

        @pl.when(i == nblk - 1)
        def _():
            g = acc_ref[...] * (1.0 / n)
            out_ref[...] = (
                jnp.dot(g, pw_ref[...], preferred_element_type=jnp.float32)
                + pb_ref[...])

    return pl.pallas_call(
        nk,
        grid=(nblk,),
        in_specs=[
            pl.BlockSpec((2, nb, d), lambda i: (0, i, 0)),
            pl.BlockSpec((2, nb, 16), lambda i: (0, i, 0)),
            pl.BlockSpec((nb, d_in), lambda i: (i, 0)),
            pl.BlockSpec((d_in, d), lambda i: (0, 0)),
            pl.BlockSpec((1, d), lambda i: (0, 0)),
            pl.BlockSpec((d, d_out), lambda i: (0, 0)),
            pl.BlockSpec((1, d_out), lambda i: (0, 0)),
        ],
        out_specs=pl.BlockSpec((1, d_out), lambda i: (0, 0)),
        out_shape=jax.ShapeDtypeStruct((1, d_out), jnp.float32),
        scratch_shapes=[pltpu.VMEM((1, d), jnp.float32)],
    )(parts, cnts, x, root, bias.reshape(1, d), proj_w, proj_b.reshape(1, d_out))


def kernel(x, edge_index, edge_attr, en1_W1, en1_b1, en1_W2, en1_b2,
           en2_W1, en2_b1, en2_W2, en2_b2, root1, bias1, root2, bias2,
           proj_W, proj_b):
    n, d_in = x.shape
    e = edge_index.shape[1]
    d_e = edge_attr.shape[1]
    h_dim = en1_W1.shape[1]

    grain = _NW * _CH
    e_pad = ((e + grain - 1) // grain) * grain
    n_acc = ((n + 1 + 127) // 128) * 128

    src = edge_index[0]
    dst = edge_index[1]
    pad = e_pad - e
    src2 = jnp.concatenate([src, jnp.zeros((pad,), jnp.int32)]).reshape(-1, _CH)
    dst2 = jnp.concatenate([dst, jnp.full((pad,), n, jnp.int32)]).reshape(-1, _CH)
    ea_p = jnp.concatenate([edge_attr, jnp.zeros((pad, d_e), jnp.float32)])

    p1_mat = en1_W2.reshape(h_dim * d_in, h_dim)
    b1_mat = en1_b2.reshape(d_in, h_dim)
    p2_mat = en2_W2.reshape(h_dim * h_dim, h_dim)
    b2_mat = en2_b2.reshape(h_dim, h_dim)

    xj1 = _gather_rows(x, src2, e_pad, d_in)
    msg1 = _edge_msgs(ea_p, xj1, en1_W1, en1_b1, p1_mat, b1_mat, e_pad)
    parts1, cnts = _scatter_rows(msg1, dst2, e_pad, h_dim, n_acc, with_cnt=True)
    h1 = _node_update(parts1[:, :n], cnts[:, :n], x, root1, bias1, n)

    xj2 = _gather_rows(h1, src2, e_pad, h_dim)
    msg2 = _edge_msgs(ea_p, xj2, en2_W1, en2_b1, p2_mat, b2_mat, e_pad)
    parts2 = _scatter_rows(msg2, dst2, e_pad, h_dim, n_acc, with_cnt=False)

    out = _node_pool_proj(parts2[:, :n], cnts[:, :n], h1, root2, bias2,
                          proj_W, proj_b, n)
    return out.reshape(-1)
